# baseline (speedup 1.0000x reference)
import functools

import jax
import jax.numpy as jnp
from jax import lax
from jax.experimental import pallas as pl
from jax.experimental.pallas import tpu as pltpu
from jax.experimental.pallas import tpu_sc as plsc

_BATCH = 16384
_HIST = 20
_DIM = 64
_VOCAB = 1000000

_info = plsc.get_sparse_core_info()
_NC, _NS = _info.num_cores, _info.num_subcores
_NW = _NC * _NS

_mesh = plsc.VectorSubcoreMesh(core_axis_name="c", subcore_axis_name="s")
_params = pltpu.CompilerParams(
    use_tc_tiling_on_sc=True, needs_layout_passes=False
)

_VR = 384
_NCH1 = 2604
assert _VR * _NCH1 == 999936
_IT1 = (_NCH1 + _NW - 1) // _NW
_TAIL0 = 999936
_FL = _DIM * _VR


@functools.partial(
    pl.kernel,
    mesh=_mesh,
    out_type=jax.ShapeDtypeStruct((_VOCAB * _DIM,), jnp.float32),
    compiler_params=_params,
    scratch_types=[
        pltpu.VMEM((_FL,), jnp.float32),
        pltpu.VMEM((_FL,), jnp.float32),
        pltpu.VMEM((_FL,), jnp.float32),
        pltpu.VMEM((_FL,), jnp.float32),
        pltpu.SemaphoreType.DMA((2,)),
        pltpu.SemaphoreType.DMA((2,)),
    ],
)
def _transpose_table(tt_hbm, tail_hbm, trm_hbm, bin0, bin1, bout0, bout1,
                     sem_i, sem_o):
    bins = (bin0, bin1)
    bouts = (bout0, bout1)
    wid = lax.axis_index("s") * _NC + lax.axis_index("c")
    iota = lax.iota(jnp.int32, 16)
    iv_init = [(iota + k * 16) * _VR for k in range(4)]

    def stage(i, b):
        c = wid + _NW * i
        v0 = c * _VR
        for d in range(_DIM):
            pltpu.async_copy(
                tt_hbm.at[d, pl.ds(v0, _VR)],
                bins[b].at[pl.ds(d * _VR, _VR)],
                sem_i.at[b],
            )

    def stage_wait(b):
        for d in range(_DIM):
            pltpu.make_async_copy(
                tt_hbm.at[0, pl.ds(0, _VR)],
                bins[b].at[pl.ds(0, _VR)],
                sem_i.at[b],
            ).wait()

    def wout(i, b):
        c = wid + _NW * i
        return pltpu.async_copy(
            bouts[b], trm_hbm.at[pl.ds(c * _FL, _FL)], sem_o.at[b]
        )

    def wout_wait(b):
        pltpu.make_async_copy(
            bouts[b], trm_hbm.at[pl.ds(0, _FL)], sem_o.at[b]
        ).wait()

    def transpose(b):
        bi, bo = bins[b], bouts[b]

        def tr_body(v, carry):
            off, i0, i1, i2, i3 = carry
            bo[pl.ds(off, 16)] = plsc.load_gather(bi, [i0])
            bo[pl.ds(off + 16, 16)] = plsc.load_gather(bi, [i1])
            bo[pl.ds(off + 32, 16)] = plsc.load_gather(bi, [i2])
            bo[pl.ds(off + 48, 16)] = plsc.load_gather(bi, [i3])
            return (off + 64, i0 + 1, i1 + 1, i2 + 1, i3 + 1)

        lax.fori_loop(0, _VR, tr_body, (0,) + tuple(iv_init))

    stage(0, 0)
    stage(1, 1)

    def chunk_body(i, carry):
        c = wid + _NW * i

        def do(b):
            @pl.when(c < _NCH1)
            def _():
                stage_wait(b)

            @pl.when(jnp.logical_and(i >= 2, c - 2 * _NW < _NCH1))
            def _():
                wout_wait(b)

            @pl.when(c < _NCH1)
            def _():
                transpose(b)
                wout(i, b)

            @pl.when(c + 2 * _NW < _NCH1)
            def _():
                stage(i + 2, b)

        @pl.when((i & 1) == 0)
        def _():
            do(0)

        @pl.when((i & 1) == 1)
        def _():
            do(1)

        return carry

    lax.fori_loop(0, _IT1, chunk_body, 0)

    def drain(i, carry):
        c = wid + _NW * i

        @pl.when(c < _NCH1)
        def _():
            @pl.when((i & 1) == 0)
            def _():
                wout_wait(0)

            @pl.when((i & 1) == 1)
            def _():
                wout_wait(1)

        return carry

    lax.fori_loop(_IT1 - 2, _IT1, drain, 0)

    @pl.when(wid == _NW - 1)
    def _():
        pltpu.sync_copy(tail_hbm, bin0.at[pl.ds(0, 4096)])
        pltpu.sync_copy(
            bin0.at[pl.ds(0, 4096)],
            trm_hbm.at[pl.ds(_TAIL0 * _DIM, 4096)],
        )


_BPW = _BATCH // _NW
_CH = 256
_NCH2 = _HIST * (_BPW // _CH)


@functools.partial(
    pl.kernel,
    mesh=_mesh,
    out_type=jax.ShapeDtypeStruct((_HIST, _DIM, _BATCH), jnp.float32),
    compiler_params=_params,
    scratch_types=[
        pltpu.VMEM((_HIST * _BPW,), jnp.int32),
        pltpu.VMEM((1, _CH), jnp.int32),
        pltpu.VMEM((1, _CH), jnp.int32),
        pltpu.VMEM((_CH,), jnp.int32),
        pltpu.VMEM((_CH,), jnp.int32),
        pltpu.VMEM((_CH, 128), jnp.float32),
        pltpu.VMEM((_CH, 128), jnp.float32),
        pltpu.VMEM((_DIM * _CH,), jnp.float32),
        pltpu.VMEM((_DIM * _CH,), jnp.float32),
        pltpu.SemaphoreType.DMA((2,)),
        pltpu.SemaphoreType.DMA((2,)),
    ],
)
def _gather_rows(xt_hbm, trm_hbm, out_hbm, idx_v, idx2_0, idx2_1,
                 lsb64_0, lsb64_1, rows0, rows1, rowst0, rowst1,
                 sem_g, sem_o):
    idx2s = (idx2_0, idx2_1)
    lsb64s = (lsb64_0, lsb64_1)
    rowss = (rows0, rows1)
    rowsts = (rowst0, rowst1)
    wid = lax.axis_index("s") * _NC + lax.axis_index("c")
    b0 = wid * _BPW
    for hh in range(_HIST):
        pltpu.async_copy(
            xt_hbm.at[hh, pl.ds(b0, _BPW)],
            idx_v.at[pl.ds(hh * _BPW, _BPW)],
            sem_g.at[0],
        )
    for hh in range(_HIST):
        pltpu.make_async_copy(
            xt_hbm.at[0, pl.ds(0, _BPW)],
            idx_v.at[pl.ds(0, _BPW)],
            sem_g.at[0],
        ).wait()
    iota = lax.iota(jnp.int32, 16)

    def prep_and_gather(t, b):
        def split_body(j, off):
            iv = idx_v[pl.ds(off, 16)]
            idx2s[b][0, pl.ds(j * 16, 16)] = lax.shift_right_logical(iv, 1)
            lsb64s[b][pl.ds(j * 16, 16)] = (iv & 1) * 64
            return off + 16

        lax.fori_loop(0, _CH // 16, split_body, t * _CH)
        return pltpu.async_copy(
            trm_hbm.at[idx2s[b].at[0]], rowss[b], sem_g.at[b]
        )

    def gather_wait(b):
        pltpu.make_async_copy(
            trm_hbm.at[idx2s[b].at[0]], rowss[b], sem_g.at[b]
        ).wait()

    def transpose(b):
        rw, rt, lb = rowss[b], rowsts[b], lsb64s[b]

        def jg_body(jg, j0):
            l64 = lb[pl.ds(j0, 16)]
            rowv = iota + j0

            def d_body(d, car):
                off, colv = car
                rt[pl.ds(off, 16)] = plsc.load_gather(rw, [rowv, colv])
                return (off + _CH, colv + 1)

            lax.fori_loop(0, _DIM, d_body, (j0, l64))
            return j0 + 16

        lax.fori_loop(0, _CH // 16, jg_body, 0)

    def wout(t, b):
        h = lax.shift_right_logical(t, 1)
        bb = b0 + (t & 1) * _CH
        for d in range(_DIM):
            pltpu.async_copy(
                rowsts[b].at[pl.ds(d * _CH, _CH)],
                out_hbm.at[h, d, pl.ds(bb, _CH)],
                sem_o.at[b],
            )

    def wout_wait(b):
        for d in range(_DIM):
            pltpu.make_async_copy(
                rowsts[b].at[pl.ds(0, _CH)],
                out_hbm.at[0, 0, pl.ds(0, _CH)],
                sem_o.at[b],
            ).wait()

    prep_and_gather(0, 0)
    prep_and_gather(1, 1)

    def chunk_body(t, carry):
        def do(b):
            gather_wait(b)

            @pl.when(t >= 2)
            def _():
                wout_wait(b)

            transpose(b)
            wout(t, b)

            @pl.when(t + 2 < _NCH2)
            def _():
                prep_and_gather(t + 2, b)

        @pl.when((t & 1) == 0)
        def _():
            do(0)

        @pl.when((t & 1) == 1)
        def _():
            do(1)

        return carry

    lax.fori_loop(0, _NCH2, chunk_body, 0)
    wout_wait(0)
    wout_wait(1)


def kernel(x, table):
    tt = table.T
    tail = table[_TAIL0:, :].reshape(4096)
    trm = _transpose_table(tt, tail).reshape(_VOCAB // 2, 128)
    out_t = _gather_rows(x.T, trm)
    return out_t.transpose(2, 0, 1)


# speedup vs baseline: 2.3564x; 2.3564x over previous
import functools

import jax
import jax.numpy as jnp
from jax import lax
from jax.experimental import pallas as pl
from jax.experimental.pallas import tpu as pltpu
from jax.experimental.pallas import tpu_sc as plsc

_BATCH = 16384
_HIST = 20
_DIM = 64

_info = plsc.get_sparse_core_info()
_NC, _NS = _info.num_cores, _info.num_subcores
_NW = _NC * _NS
_BPW = _BATCH // _NW
_NBUF = 3


@functools.partial(
    pl.kernel,
    mesh=plsc.VectorSubcoreMesh(core_axis_name="c", subcore_axis_name="s"),
    out_type=jax.ShapeDtypeStruct((_BATCH, _HIST, _DIM), jnp.float32),
    compiler_params=pltpu.CompilerParams(use_tc_tiling_on_sc=False),
    scratch_types=[
        pltpu.VMEM((_HIST, _BPW), jnp.int32),
        pltpu.VMEM((_NBUF, _BPW, _DIM), jnp.float32),
        pltpu.SemaphoreType.DMA((_NBUF,)),
        pltpu.SemaphoreType.DMA((_NBUF,)),
    ],
)
def _gather_rows(xt_hbm, table_hbm, out_hbm, idx_v, rows, sem_g, sem_o):
    wid = lax.axis_index("s") * _NC + lax.axis_index("c")
    b0 = wid * _BPW
    pltpu.sync_copy(xt_hbm.at[:, pl.ds(b0, _BPW)], idx_v)

    def gather(h):
        b = h % _NBUF
        return pltpu.async_copy(
            table_hbm.at[idx_v.at[h]], rows.at[b], sem_g.at[b]
        )

    def writeout(h):
        b = h % _NBUF
        return pltpu.async_copy(
            rows.at[b],
            out_hbm.at[pl.ds(b0, _BPW), h],
            sem_o.at[b],
        )

    cp_g = [None] * _HIST
    cp_o = [None] * _HIST
    for b in range(_NBUF):
        cp_g[b] = gather(b)
    for h in range(_HIST):
        if h > 0:
            cp_o[h - 1].wait()
            nxt = h - 1 + _NBUF
            if nxt < _HIST:
                cp_g[nxt] = gather(nxt)
        cp_g[h].wait()
        cp_o[h] = writeout(h)
    cp_o[_HIST - 1].wait()


def kernel(x, table):
    return _gather_rows(x.T, table)
